# Initial kernel scaffold; baseline (speedup 1.0000x reference)
import jax
import jax.numpy as jnp
from jax.experimental import pallas as pl


def kernel(ffn_out, lengths, W, b, lora_A, lora_B):
    raise NotImplementedError("write your pallas kernel here")


# speedup vs baseline: 5.2393x; 5.2393x over previous
import functools

import jax
import jax.numpy as jnp
from jax import lax
from jax.experimental import pallas as pl
from jax.experimental.pallas import tpu as pltpu
from jax.experimental.pallas import tpu_sc as plsc

ALPHA = 32.0


def _sc_gather(table, flat_idx):
    n, d = flat_idx.shape[0], table.shape[1]
    info = plsc.get_sparse_core_info()
    nw = info.num_cores * info.num_subcores
    n_per_w = n // nw
    mesh = plsc.VectorSubcoreMesh(core_axis_name="c", subcore_axis_name="s")

    @functools.partial(
        pl.kernel,
        mesh=mesh,
        out_type=jax.ShapeDtypeStruct((n, d), jnp.float32),
        scratch_types=[
            pltpu.VMEM((n_per_w,), jnp.int32),
            pltpu.VMEM((n_per_w, d), jnp.float32),
            pltpu.SemaphoreType.DMA,
        ],
    )
    def gather_kernel(table_hbm, idx_hbm, out_hbm, idx_v, rows_v, sem):
        wid = lax.axis_index("s") * info.num_cores + lax.axis_index("c")
        base = wid * n_per_w
        pltpu.sync_copy(idx_hbm.at[pl.ds(base, n_per_w)], idx_v)
        pltpu.async_copy(table_hbm.at[idx_v], rows_v, sem).wait()
        pltpu.sync_copy(rows_v, out_hbm.at[pl.ds(base, n_per_w)])

    return gather_kernel(table, flat_idx)


def _tc_lora_linear(x, w, b2d, lora_a, lora_b, scaling):
    n, d = x.shape

    def body(x_ref, w_ref, b_ref, a_ref, bb_ref, o_ref):
        m = w_ref[:] + scaling * jnp.dot(
            bb_ref[:], a_ref[:], preferred_element_type=jnp.float32
        )
        o_ref[:] = (
            lax.dot_general(
                x_ref[:], m, (((1,), (1,)), ((), ())),
                preferred_element_type=jnp.float32,
            )
            + b_ref[:]
        )

    return pl.pallas_call(
        body,
        out_shape=jax.ShapeDtypeStruct((n, d), jnp.float32),
    )(x, w, b2d, lora_a, lora_b)


def kernel(ffn_out, lengths, W, b, lora_A, lora_B):
    bsz, s, i, d = ffn_out.shape
    r = lora_A.shape[0]
    scaling = ALPHA / r

    table = ffn_out.reshape(bsz * s * i, d)
    flat_idx = (
        jnp.arange(bsz * s, dtype=jnp.int32) * i
        + lengths.reshape(-1).astype(jnp.int32)
    )
    x = _sc_gather(table, flat_idx)
    out = _tc_lora_linear(x, W, b.reshape(1, d), lora_A, lora_B, scaling)
    return out.reshape(bsz, s, d)
